# baseline (speedup 1.0000x reference)
import functools

import jax
import jax.numpy as jnp
from jax import lax
from jax.experimental import pallas as pl
from jax.experimental.pallas import tpu as pltpu
from jax.experimental.pallas import tpu_sc as plsc

N = 25000
FEAT = 128
H = 256
E = 400000
LBL = 100000

NC = 2
NS = 16
NW = NC * NS
ET = E // NS
ETP = 25600
NSB = 64

NSL = 2
SW = H // NSL
SPAD = 25600
NH = SPAD // 2
ACCR = NH + 128
CACC = 25728
TRASH = SPAD
CWID = 16

LTW = 3128
LPAD = LTW * NW
GBS = [112] * 27 + [104]

_mesh = lambda: plsc.VectorSubcoreMesh(
    core_axis_name="c", subcore_axis_name="s", num_cores=NC, num_subcores=NS)


def _enc_body(x_ref, w_ref, b_ref, emb_ref, *o_refs):
    o = (lax.dot_general(x_ref[...], w_ref[...], (((1,), (1,)), ((), ())),
                         preferred_element_type=jnp.float32)
         + b_ref[...] + emb_ref[...])
    for k in range(NSL):
        o_refs[k][...] = o[:, k * SW:(k + 1) * SW]


def _encode(x, w, b, emb):
    rb = 1000
    return pl.pallas_call(
        _enc_body,
        grid=(N // rb,),
        in_specs=[
            pl.BlockSpec((rb, FEAT), lambda i: (i, 0)),
            pl.BlockSpec((H, FEAT), lambda i: (0, 0)),
            pl.BlockSpec((1, H), lambda i: (0, 0)),
            pl.BlockSpec((rb, H), lambda i: (i, 0)),
        ],
        out_specs=[pl.BlockSpec((rb, SW), lambda i: (i, 0))] * NSL,
        out_shape=[jax.ShapeDtypeStruct((N, SW), jnp.float32)] * NSL,
    )(x, w, b.reshape(1, H), emb)


def _combine_body(*refs, relu, split):
    s_refs = refs[:NSL]
    c_ref = refs[NSL]
    x_refs = refs[NSL + 1:2 * NSL + 1]
    wl_ref, bl_ref, wr_ref = refs[2 * NSL + 1:2 * NSL + 4]
    o_refs = refs[2 * NSL + 4:]
    s = jnp.concatenate([r[...] for r in s_refs], axis=1)
    x = jnp.concatenate([r[...] for r in x_refs], axis=1)
    r = 1.0 / jnp.maximum(c_ref[..., 0:1], 1.0)
    agg = s * r
    o = (lax.dot_general(agg, wl_ref[...], (((1,), (1,)), ((), ())),
                         preferred_element_type=jnp.float32)
         + bl_ref[...]
         + lax.dot_general(x, wr_ref[...], (((1,), (1,)), ((), ())),
                           preferred_element_type=jnp.float32))
    if relu:
        o = jnp.maximum(o, 0.0)
    if split:
        for k in range(NSL):
            o_refs[k][...] = o[:, k * SW:(k + 1) * SW]
    else:
        o_refs[0][...] = o


def _combine(s_slices, c, x_slices, wl, bl, wr, relu, split):
    rb = 1000
    if split:
        out_specs = [pl.BlockSpec((rb, SW), lambda i: (i, 0))] * NSL
        out_shape = [jax.ShapeDtypeStruct((N, SW), jnp.float32)] * NSL
    else:
        out_specs = [pl.BlockSpec((rb, H), lambda i: (i, 0))]
        out_shape = [jax.ShapeDtypeStruct((N, H), jnp.float32)]
    return pl.pallas_call(
        functools.partial(_combine_body, relu=relu, split=split),
        grid=(N // rb,),
        in_specs=(
            [pl.BlockSpec((rb, SW), lambda i: (i, 0))] * NSL
            + [pl.BlockSpec((rb, SW), lambda i: (i, 0))]
            + [pl.BlockSpec((rb, SW), lambda i: (i, 0))] * NSL
            + [pl.BlockSpec((H, H), lambda i: (0, 0)),
               pl.BlockSpec((1, H), lambda i: (0, 0)),
               pl.BlockSpec((H, H), lambda i: (0, 0))]
        ),
        out_specs=out_specs,
        out_shape=out_shape,
    )(*s_slices, c, *x_slices, wl, bl.reshape(1, H), wr)


def _dot_body(a_ref, b_ref, o_ref):
    o_ref[...] = jnp.sum(a_ref[...] * b_ref[...], axis=1, keepdims=True)


def _edge_dot(a, b):
    rb = LTW
    return pl.pallas_call(
        _dot_body,
        grid=(LPAD // rb,),
        in_specs=[
            pl.BlockSpec((rb, H), lambda i: (i, 0)),
            pl.BlockSpec((rb, H), lambda i: (i, 0)),
        ],
        out_specs=pl.BlockSpec((rb, 1), lambda i: (i, 0)),
        out_shape=jax.ShapeDtypeStruct((LPAD, 1), jnp.float32),
    )(a, b)


def _zero2d(ref, nrows, ncols):
    zeros16 = jnp.zeros((16,), jnp.float32)
    for colblk in range(ncols // 16):
        def body(i, carry, _c=colblk):
            ref[i, pl.ds(_c * 16, 16)] = zeros16
            return carry
        lax.fori_loop(0, nrows, body, 0)


NJB = ETP // NSB


def _segsum_body(*refs):
    xs = refs[:NSL]
    g_hbm, s_hbm = refs[NSL], refs[NSL + 1]
    outs = refs[NSL + 2:2 * NSL + 2]
    (gst0, gst1, ssr0, ssr1, ssm0, ssm1, rows0, rows1, zb, acc,
     sem0, sem1) = refs[2 * NSL + 2:]
    cid = lax.axis_index("c")
    sid = lax.axis_index("s")
    gst = (gst0, gst1)
    ssr = (ssr0, ssr1)
    ssm = (ssm0, ssm1)
    rows = (rows0, rows1)
    sems = (sem0, sem1)

    _zero2d(zb, 40, SW)
    eb = sid * ETP

    for f in range(NSL):
        for nh in range(2):
            def run_pass(_f=f, _nh=nh):
                xk = xs[_f]
                ok = outs[_f]
                base = _nh * NH
                r0 = sid * (ACCR // NS)
                off = 0
                for n in [40] * 20 + [8]:
                    pltpu.sync_copy(zb.at[pl.ds(0, n)],
                                    acc.at[pl.ds(r0 + off, n)])
                    off += n
                plsc.subcore_barrier()

                def localize(b):
                    for q in range(NSB // 16):
                        s = ssr[b][pl.ds(q * 16, 16)]
                        loc = s - base
                        m = (loc >= 0) & (loc < NH)
                        ssm[b][pl.ds(q * 16, 16)] = jnp.where(m, loc, NH)

                def pair(j2, carry):
                    b0 = eb + 2 * j2 * NSB
                    pltpu.sync_copy(g_hbm.at[pl.ds(b0, NSB)], gst[0])
                    pltpu.sync_copy(s_hbm.at[pl.ds(b0, NSB)], ssr[0])
                    cp0 = pltpu.async_copy(xk.at[gst[0]], rows[0], sems[0])
                    pltpu.sync_copy(g_hbm.at[pl.ds(b0 + NSB, NSB)], gst[1])
                    pltpu.sync_copy(s_hbm.at[pl.ds(b0 + NSB, NSB)], ssr[1])
                    localize(0)
                    cp1 = pltpu.async_copy(xk.at[gst[1]], rows[1], sems[1])
                    cp0.wait()
                    pltpu.sync_copy(rows[0], acc.at[ssm[0]], add=True)
                    localize(1)
                    cp1.wait()
                    pltpu.sync_copy(rows[1], acc.at[ssm[1]], add=True)
                    return carry
                lax.fori_loop(0, NJB // 2, pair, 0)
                plsc.subcore_barrier()
                w0 = sid * (NH // NS)
                pltpu.sync_copy(acc.at[pl.ds(w0, NH // NS)],
                                ok.at[pl.ds(base + w0, NH // NS)])
                plsc.subcore_barrier()
            pl.when(cid == f)(run_pass)


def _sc_segsum(x_slices, g1, s1):
    k = pl.kernel(
        _segsum_body,
        out_type=tuple(jax.ShapeDtypeStruct((SPAD, SW), jnp.float32)
                       for _ in range(NSL)),
        mesh=_mesh(),
        scratch_types=[
            pltpu.VMEM((NSB,), jnp.int32),
            pltpu.VMEM((NSB,), jnp.int32),
            pltpu.VMEM((NSB,), jnp.int32),
            pltpu.VMEM((NSB,), jnp.int32),
            pltpu.VMEM((NSB,), jnp.int32),
            pltpu.VMEM((NSB,), jnp.int32),
            pltpu.VMEM((NSB, SW), jnp.float32),
            pltpu.VMEM((NSB, SW), jnp.float32),
            pltpu.VMEM((40, SW), jnp.float32),
            pltpu.VMEM_SHARED((ACCR, SW), jnp.float32),
            pltpu.SemaphoreType.DMA,
            pltpu.SemaphoreType.DMA,
        ],
    )
    return k(*x_slices, g1, s1)


def _counts_body(su_hbm, sd_hbm, cu_hbm, cp_hbm,
                 sst0, sst1, ssm0, ssm1, ones, zb, cnt):
    cid = lax.axis_index("c")
    sid = lax.axis_index("s")
    lanes = lax.iota(jnp.int32, 16)
    e0 = jnp.where(lanes == 0, 1.0, 0.0)
    zeros16 = jnp.zeros((16,), jnp.float32)
    sst = (sst0, sst1)
    ssm = (ssm0, ssm1)

    _zero2d(zb, 40, SW)

    def fill_ones(i, carry):
        ones[i, pl.ds(0, 16)] = e0
        for cb in range(1, SW // 16):
            ones[i, pl.ds(cb * 16, 16)] = zeros16
        return carry
    lax.fori_loop(0, NSB, fill_ones, 0)

    eb = sid * ETP

    for d in range(2):
        for nh in range(2):
            def run_pass(_d=d, _nh=nh):
                s_hbm = (su_hbm, sd_hbm)[_d]
                out_hbm = (cu_hbm, cp_hbm)[_d]
                base = _nh * NH
                r0 = sid * (ACCR // NS)
                off = 0
                for n in [40] * 20 + [8]:
                    pltpu.sync_copy(zb.at[pl.ds(0, n)],
                                    cnt.at[pl.ds(r0 + off, n)])
                    off += n
                plsc.subcore_barrier()

                def localize(b):
                    for q in range(NSB // 16):
                        s = sst[b][pl.ds(q * 16, 16)]
                        loc = s - base
                        m = (loc >= 0) & (loc < NH)
                        ssm[b][pl.ds(q * 16, 16)] = jnp.where(m, loc, NH)

                def pair(j2, carry):
                    b0 = eb + 2 * j2 * NSB
                    pltpu.sync_copy(s_hbm.at[pl.ds(b0, NSB)], sst[0])
                    pltpu.sync_copy(s_hbm.at[pl.ds(b0 + NSB, NSB)], sst[1])
                    localize(0)
                    localize(1)
                    pltpu.sync_copy(ones, cnt.at[ssm[0]], add=True)
                    pltpu.sync_copy(ones, cnt.at[ssm[1]], add=True)
                    return carry
                lax.fori_loop(0, NJB // 2, pair, 0)
                plsc.subcore_barrier()
                w0 = sid * (NH // NS)
                pltpu.sync_copy(cnt.at[pl.ds(w0, NH // NS)],
                                out_hbm.at[pl.ds(base + w0, NH // NS)])
                plsc.subcore_barrier()
            pl.when(cid == d)(run_pass)


def _sc_counts(s4u, s4p):
    k = pl.kernel(
        _counts_body,
        out_type=(jax.ShapeDtypeStruct((SPAD, SW), jnp.float32),
                  jax.ShapeDtypeStruct((SPAD, SW), jnp.float32)),
        mesh=_mesh(),
        scratch_types=[
            pltpu.VMEM((NSB,), jnp.int32),
            pltpu.VMEM((NSB,), jnp.int32),
            pltpu.VMEM((NSB,), jnp.int32),
            pltpu.VMEM((NSB,), jnp.int32),
            pltpu.VMEM((NSB, SW), jnp.float32),
            pltpu.VMEM((40, SW), jnp.float32),
            pltpu.VMEM_SHARED((ACCR, SW), jnp.float32),
        ],
    )
    return k(s4u, s4p)


def _gather_body(hu_hbm, hp_hbm, i0_hbm, i1_hbm, eu_hbm, ep_hbm,
                 iv0, iv1, rows0, rows1, sem0, sem1):
    cid = lax.axis_index("c")
    sid = lax.axis_index("s")
    wid = cid * NS + sid
    base = wid * LTW
    pltpu.sync_copy(i0_hbm.at[pl.ds(base, LTW)], iv0)
    pltpu.sync_copy(i1_hbm.at[pl.ds(base, LTW)], iv1)
    boff = 0
    for gb in GBS:
        cp0 = pltpu.async_copy(hu_hbm.at[iv0.at[pl.ds(boff, gb)]],
                               rows0.at[pl.ds(0, gb)], sem0)
        cp1 = pltpu.async_copy(hp_hbm.at[iv1.at[pl.ds(boff, gb)]],
                               rows1.at[pl.ds(0, gb)], sem1)
        cp0.wait()
        cp1.wait()
        pltpu.sync_copy(rows0.at[pl.ds(0, gb)],
                        eu_hbm.at[pl.ds(base + boff, gb)])
        pltpu.sync_copy(rows1.at[pl.ds(0, gb)],
                        ep_hbm.at[pl.ds(base + boff, gb)])
        boff += gb


def _sc_gather2(hu, hp, i0, i1):
    k = pl.kernel(
        _gather_body,
        out_type=(jax.ShapeDtypeStruct((LPAD, H), jnp.float32),
                  jax.ShapeDtypeStruct((LPAD, H), jnp.float32)),
        mesh=_mesh(),
        scratch_types=[
            pltpu.VMEM((LTW,), jnp.int32),
            pltpu.VMEM((LTW,), jnp.int32),
            pltpu.VMEM((GBS[0], H), jnp.float32),
            pltpu.VMEM((GBS[0], H), jnp.float32),
            pltpu.SemaphoreType.DMA,
            pltpu.SemaphoreType.DMA,
        ],
    )
    return k(hu, hp, i0, i1)


def _edges4(v, pad):
    v = v.astype(jnp.int32).reshape(NS, ET)
    v = jnp.concatenate(
        [v, jnp.full((NS, ETP - ET), pad, jnp.int32)], axis=1)
    return v.reshape(-1)


def kernel(x_user, x_product, node_id_user, node_id_product, edge_index_up,
           edge_label_index, W_user_lin, b_user_lin, emb_user, W_prod_lin,
           b_prod_lin, emb_prod, Wl_up, bl_up, Wr_up, Wl_pu, bl_pu, Wr_pu):
    hu_s = _encode(x_user, W_user_lin, b_user_lin, emb_user)
    hp_s = _encode(x_product, W_prod_lin, b_prod_lin, emb_prod)

    src4 = _edges4(edge_index_up[0], TRASH)
    dst4 = _edges4(edge_index_up[1], TRASH)
    srcg = _edges4(edge_index_up[0], 0)
    dstg = _edges4(edge_index_up[1], 0)

    cu, cp = _sc_counts(src4, dst4)

    su_s = _sc_segsum(hp_s, dstg, src4)
    sp_s = _sc_segsum(hu_s, srcg, dst4)
    h1u_s = _combine(su_s, cu, hu_s, Wl_pu, bl_pu, Wr_pu, relu=True, split=True)
    h1p_s = _combine(sp_s, cp, hp_s, Wl_up, bl_up, Wr_up, relu=True, split=True)

    su2_s = _sc_segsum(h1p_s, dstg, src4)
    sp2_s = _sc_segsum(h1u_s, srcg, dst4)
    (h2u,) = _combine(su2_s, cu, h1u_s, Wl_pu, bl_pu, Wr_pu,
                      relu=False, split=False)
    (h2p,) = _combine(sp2_s, cp, h1p_s, Wl_up, bl_up, Wr_up,
                      relu=False, split=False)

    i0 = jnp.pad(edge_label_index[0].astype(jnp.int32), (0, LPAD - LBL))
    i1 = jnp.pad(edge_label_index[1].astype(jnp.int32), (0, LPAD - LBL))
    eu, ep = _sc_gather2(h2u, h2p, i0, i1)
    return _edge_dot(eu, ep).reshape(LPAD)[:LBL]


# speedup vs baseline: 1.2968x; 1.2968x over previous
import functools

import jax
import jax.numpy as jnp
from jax import lax
from jax.experimental import pallas as pl
from jax.experimental.pallas import tpu as pltpu
from jax.experimental.pallas import tpu_sc as plsc

N = 25000
FEAT = 128
H = 256
E = 400000
LBL = 100000

NC = 2
NS = 16
NW = NC * NS
ET = E // NS
ETP = 25600
NSB = 64

NSL = 2
SW = H // NSL
SPAD = 25600
NH = SPAD // 2
ACCR = NH + 128
CACC = 25728
TRASH = SPAD
CWID = 16

LTW = 3128
LPAD = LTW * NW
GBS = [112] * 27 + [104]

_mesh = lambda: plsc.VectorSubcoreMesh(
    core_axis_name="c", subcore_axis_name="s", num_cores=NC, num_subcores=NS)


def _enc_body(x_ref, w_ref, b_ref, emb_ref, *o_refs):
    o = (lax.dot_general(x_ref[...], w_ref[...], (((1,), (1,)), ((), ())),
                         preferred_element_type=jnp.float32)
         + b_ref[...] + emb_ref[...])
    for k in range(NSL):
        o_refs[k][...] = o[:, k * SW:(k + 1) * SW]


def _encode(x, w, b, emb):
    rb = 1000
    return pl.pallas_call(
        _enc_body,
        grid=(N // rb,),
        in_specs=[
            pl.BlockSpec((rb, FEAT), lambda i: (i, 0)),
            pl.BlockSpec((H, FEAT), lambda i: (0, 0)),
            pl.BlockSpec((1, H), lambda i: (0, 0)),
            pl.BlockSpec((rb, H), lambda i: (i, 0)),
        ],
        out_specs=[pl.BlockSpec((rb, SW), lambda i: (i, 0))] * NSL,
        out_shape=[jax.ShapeDtypeStruct((N, SW), jnp.float32)] * NSL,
    )(x, w, b.reshape(1, H), emb)


def _combine_body(*refs, relu, split):
    s_refs = refs[:NSL]
    c_ref = refs[NSL]
    x_refs = refs[NSL + 1:2 * NSL + 1]
    wl_ref, bl_ref, wr_ref = refs[2 * NSL + 1:2 * NSL + 4]
    o_refs = refs[2 * NSL + 4:]
    s = jnp.concatenate([r[...] for r in s_refs], axis=1)
    x = jnp.concatenate([r[...] for r in x_refs], axis=1)
    r = 1.0 / jnp.maximum(c_ref[..., 0:1], 1.0)
    agg = s * r
    o = (lax.dot_general(agg, wl_ref[...], (((1,), (1,)), ((), ())),
                         preferred_element_type=jnp.float32)
         + bl_ref[...]
         + lax.dot_general(x, wr_ref[...], (((1,), (1,)), ((), ())),
                           preferred_element_type=jnp.float32))
    if relu:
        o = jnp.maximum(o, 0.0)
    if split:
        for k in range(NSL):
            o_refs[k][...] = o[:, k * SW:(k + 1) * SW]
    else:
        o_refs[0][...] = o


def _combine(s_slices, c, x_slices, wl, bl, wr, relu, split):
    rb = 1000
    if split:
        out_specs = [pl.BlockSpec((rb, SW), lambda i: (i, 0))] * NSL
        out_shape = [jax.ShapeDtypeStruct((N, SW), jnp.float32)] * NSL
    else:
        out_specs = [pl.BlockSpec((rb, H), lambda i: (i, 0))]
        out_shape = [jax.ShapeDtypeStruct((N, H), jnp.float32)]
    return pl.pallas_call(
        functools.partial(_combine_body, relu=relu, split=split),
        grid=(N // rb,),
        in_specs=(
            [pl.BlockSpec((rb, SW), lambda i: (i, 0))] * NSL
            + [pl.BlockSpec((rb, SW), lambda i: (i, 0))]
            + [pl.BlockSpec((rb, SW), lambda i: (i, 0))] * NSL
            + [pl.BlockSpec((H, H), lambda i: (0, 0)),
               pl.BlockSpec((1, H), lambda i: (0, 0)),
               pl.BlockSpec((H, H), lambda i: (0, 0))]
        ),
        out_specs=out_specs,
        out_shape=out_shape,
    )(*s_slices, c, *x_slices, wl, bl.reshape(1, H), wr)


def _dot_body(a_ref, b_ref, o_ref):
    o_ref[...] = jnp.sum(a_ref[...] * b_ref[...], axis=1, keepdims=True)


def _edge_dot(a, b):
    rb = LTW
    return pl.pallas_call(
        _dot_body,
        grid=(LPAD // rb,),
        in_specs=[
            pl.BlockSpec((rb, H), lambda i: (i, 0)),
            pl.BlockSpec((rb, H), lambda i: (i, 0)),
        ],
        out_specs=pl.BlockSpec((rb, 1), lambda i: (i, 0)),
        out_shape=jax.ShapeDtypeStruct((LPAD, 1), jnp.float32),
    )(a, b)


def _zero2d(ref, nrows, ncols):
    zeros16 = jnp.zeros((16,), jnp.float32)
    for colblk in range(ncols // 16):
        def body(i, carry, _c=colblk):
            ref[i, pl.ds(_c * 16, 16)] = zeros16
            return carry
        lax.fori_loop(0, nrows, body, 0)


NJB = ETP // NSB


def _segsum_body(*refs):
    xs = refs[:NSL]
    g_hbm, s_hbm = refs[NSL], refs[NSL + 1]
    outs = refs[NSL + 2:2 * NSL + 2]
    (gstbig, sstbig, ssm0, ssm1, rows0, rows1, zb, acc,
     sem0, sem1) = refs[2 * NSL + 2:]
    cid = lax.axis_index("c")
    sid = lax.axis_index("s")
    ssm = (ssm0, ssm1)
    rows = (rows0, rows1)
    sems = (sem0, sem1)

    _zero2d(zb, 40, SW)
    eb = sid * ETP

    for f in range(NSL):
        for nh in range(2):
            def run_pass(_f=f, _nh=nh):
                xk = xs[_f]
                ok = outs[_f]
                base = _nh * NH
                r0 = sid * (ACCR // NS)
                off = 0
                for n in [40] * 20 + [8]:
                    pltpu.sync_copy(zb.at[pl.ds(0, n)],
                                    acc.at[pl.ds(r0 + off, n)])
                    off += n
                plsc.subcore_barrier()

                def localize(j):
                    for q in range(NSB // 16):
                        s = sstbig[pl.ds(j * NSB + q * 16, 16)]
                        loc = s - base
                        m = (loc >= 0) & (loc < NH)
                        ssm[j % 2][pl.ds(q * 16, 16)] = jnp.where(m, loc, NH)

                def blk(b2, carry):
                    b0 = eb + b2 * 16 * NSB
                    pltpu.sync_copy(g_hbm.at[pl.ds(b0, 16 * NSB)], gstbig)
                    pltpu.sync_copy(s_hbm.at[pl.ds(b0, 16 * NSB)], sstbig)
                    cp = pltpu.async_copy(
                        xk.at[gstbig.at[pl.ds(0, NSB)]], rows[0], sems[0])
                    for j in range(16):
                        localize(j)
                        cpn = None
                        if j + 1 < 16:
                            cpn = pltpu.async_copy(
                                xk.at[gstbig.at[pl.ds((j + 1) * NSB, NSB)]],
                                rows[(j + 1) % 2], sems[(j + 1) % 2])
                        cp.wait()
                        pltpu.sync_copy(rows[j % 2], acc.at[ssm[j % 2]],
                                        add=True)
                        cp = cpn
                    return carry
                lax.fori_loop(0, NJB // 16, blk, 0)
                plsc.subcore_barrier()
                w0 = sid * (NH // NS)
                pltpu.sync_copy(acc.at[pl.ds(w0, NH // NS)],
                                ok.at[pl.ds(base + w0, NH // NS)])
                plsc.subcore_barrier()
            pl.when(cid == f)(run_pass)


def _sc_segsum(x_slices, g1, s1):
    k = pl.kernel(
        _segsum_body,
        out_type=tuple(jax.ShapeDtypeStruct((SPAD, SW), jnp.float32)
                       for _ in range(NSL)),
        mesh=_mesh(),
        scratch_types=[
            pltpu.VMEM((16 * NSB,), jnp.int32),
            pltpu.VMEM((16 * NSB,), jnp.int32),
            pltpu.VMEM((NSB,), jnp.int32),
            pltpu.VMEM((NSB,), jnp.int32),
            pltpu.VMEM((NSB, SW), jnp.float32),
            pltpu.VMEM((NSB, SW), jnp.float32),
            pltpu.VMEM((40, SW), jnp.float32),
            pltpu.VMEM_SHARED((ACCR, SW), jnp.float32),
            pltpu.SemaphoreType.DMA,
            pltpu.SemaphoreType.DMA,
        ],
    )
    return k(*x_slices, g1, s1)


def _counts_body(su_hbm, sd_hbm, cu_hbm, cp_hbm,
                 sst0, sst1, ssm0, ssm1, ones, zb, cnt):
    cid = lax.axis_index("c")
    sid = lax.axis_index("s")
    lanes = lax.iota(jnp.int32, 16)
    e0 = jnp.where(lanes == 0, 1.0, 0.0)
    zeros16 = jnp.zeros((16,), jnp.float32)
    sst = (sst0, sst1)
    ssm = (ssm0, ssm1)

    _zero2d(zb, 40, SW)

    def fill_ones(i, carry):
        ones[i, pl.ds(0, 16)] = e0
        for cb in range(1, SW // 16):
            ones[i, pl.ds(cb * 16, 16)] = zeros16
        return carry
    lax.fori_loop(0, NSB, fill_ones, 0)

    eb = sid * ETP

    for d in range(2):
        for nh in range(2):
            def run_pass(_d=d, _nh=nh):
                s_hbm = (su_hbm, sd_hbm)[_d]
                out_hbm = (cu_hbm, cp_hbm)[_d]
                base = _nh * NH
                r0 = sid * (ACCR // NS)
                off = 0
                for n in [40] * 20 + [8]:
                    pltpu.sync_copy(zb.at[pl.ds(0, n)],
                                    cnt.at[pl.ds(r0 + off, n)])
                    off += n
                plsc.subcore_barrier()

                def localize(b):
                    for q in range(NSB // 16):
                        s = sst[b][pl.ds(q * 16, 16)]
                        loc = s - base
                        m = (loc >= 0) & (loc < NH)
                        ssm[b][pl.ds(q * 16, 16)] = jnp.where(m, loc, NH)

                def pair(j2, carry):
                    b0 = eb + 2 * j2 * NSB
                    pltpu.sync_copy(s_hbm.at[pl.ds(b0, NSB)], sst[0])
                    pltpu.sync_copy(s_hbm.at[pl.ds(b0 + NSB, NSB)], sst[1])
                    localize(0)
                    localize(1)
                    pltpu.sync_copy(ones, cnt.at[ssm[0]], add=True)
                    pltpu.sync_copy(ones, cnt.at[ssm[1]], add=True)
                    return carry
                lax.fori_loop(0, NJB // 2, pair, 0)
                plsc.subcore_barrier()
                w0 = sid * (NH // NS)
                pltpu.sync_copy(cnt.at[pl.ds(w0, NH // NS)],
                                out_hbm.at[pl.ds(base + w0, NH // NS)])
                plsc.subcore_barrier()
            pl.when(cid == d)(run_pass)


def _sc_counts(s4u, s4p):
    k = pl.kernel(
        _counts_body,
        out_type=(jax.ShapeDtypeStruct((SPAD, SW), jnp.float32),
                  jax.ShapeDtypeStruct((SPAD, SW), jnp.float32)),
        mesh=_mesh(),
        scratch_types=[
            pltpu.VMEM((NSB,), jnp.int32),
            pltpu.VMEM((NSB,), jnp.int32),
            pltpu.VMEM((NSB,), jnp.int32),
            pltpu.VMEM((NSB,), jnp.int32),
            pltpu.VMEM((NSB, SW), jnp.float32),
            pltpu.VMEM((40, SW), jnp.float32),
            pltpu.VMEM_SHARED((ACCR, SW), jnp.float32),
        ],
    )
    return k(s4u, s4p)


def _gather_body(hu_hbm, hp_hbm, i0_hbm, i1_hbm, eu_hbm, ep_hbm,
                 iv0, iv1, rows0, rows1, sem0, sem1):
    cid = lax.axis_index("c")
    sid = lax.axis_index("s")
    wid = cid * NS + sid
    base = wid * LTW
    pltpu.sync_copy(i0_hbm.at[pl.ds(base, LTW)], iv0)
    pltpu.sync_copy(i1_hbm.at[pl.ds(base, LTW)], iv1)
    boff = 0
    for gb in GBS:
        cp0 = pltpu.async_copy(hu_hbm.at[iv0.at[pl.ds(boff, gb)]],
                               rows0.at[pl.ds(0, gb)], sem0)
        cp1 = pltpu.async_copy(hp_hbm.at[iv1.at[pl.ds(boff, gb)]],
                               rows1.at[pl.ds(0, gb)], sem1)
        cp0.wait()
        cp1.wait()
        pltpu.sync_copy(rows0.at[pl.ds(0, gb)],
                        eu_hbm.at[pl.ds(base + boff, gb)])
        pltpu.sync_copy(rows1.at[pl.ds(0, gb)],
                        ep_hbm.at[pl.ds(base + boff, gb)])
        boff += gb


def _sc_gather2(hu, hp, i0, i1):
    k = pl.kernel(
        _gather_body,
        out_type=(jax.ShapeDtypeStruct((LPAD, H), jnp.float32),
                  jax.ShapeDtypeStruct((LPAD, H), jnp.float32)),
        mesh=_mesh(),
        scratch_types=[
            pltpu.VMEM((LTW,), jnp.int32),
            pltpu.VMEM((LTW,), jnp.int32),
            pltpu.VMEM((GBS[0], H), jnp.float32),
            pltpu.VMEM((GBS[0], H), jnp.float32),
            pltpu.SemaphoreType.DMA,
            pltpu.SemaphoreType.DMA,
        ],
    )
    return k(hu, hp, i0, i1)


def _edges4(v, pad):
    v = v.astype(jnp.int32).reshape(NS, ET)
    v = jnp.concatenate(
        [v, jnp.full((NS, ETP - ET), pad, jnp.int32)], axis=1)
    return v.reshape(-1)


def kernel(x_user, x_product, node_id_user, node_id_product, edge_index_up,
           edge_label_index, W_user_lin, b_user_lin, emb_user, W_prod_lin,
           b_prod_lin, emb_prod, Wl_up, bl_up, Wr_up, Wl_pu, bl_pu, Wr_pu):
    hu_s = _encode(x_user, W_user_lin, b_user_lin, emb_user)
    hp_s = _encode(x_product, W_prod_lin, b_prod_lin, emb_prod)

    src4 = _edges4(edge_index_up[0], TRASH)
    dst4 = _edges4(edge_index_up[1], TRASH)
    srcg = _edges4(edge_index_up[0], 0)
    dstg = _edges4(edge_index_up[1], 0)

    cu, cp = _sc_counts(src4, dst4)

    su_s = _sc_segsum(hp_s, dstg, src4)
    sp_s = _sc_segsum(hu_s, srcg, dst4)
    h1u_s = _combine(su_s, cu, hu_s, Wl_pu, bl_pu, Wr_pu, relu=True, split=True)
    h1p_s = _combine(sp_s, cp, hp_s, Wl_up, bl_up, Wr_up, relu=True, split=True)

    su2_s = _sc_segsum(h1p_s, dstg, src4)
    sp2_s = _sc_segsum(h1u_s, srcg, dst4)
    (h2u,) = _combine(su2_s, cu, h1u_s, Wl_pu, bl_pu, Wr_pu,
                      relu=False, split=False)
    (h2p,) = _combine(sp2_s, cp, h1p_s, Wl_up, bl_up, Wr_up,
                      relu=False, split=False)

    i0 = jnp.pad(edge_label_index[0].astype(jnp.int32), (0, LPAD - LBL))
    i1 = jnp.pad(edge_label_index[1].astype(jnp.int32), (0, LPAD - LBL))
    eu, ep = _sc_gather2(h2u, h2p, i0, i1)
    return _edge_dot(eu, ep).reshape(LPAD)[:LBL]


# speedup vs baseline: 1.3228x; 1.0200x over previous
import functools

import jax
import jax.numpy as jnp
from jax import lax
from jax.experimental import pallas as pl
from jax.experimental.pallas import tpu as pltpu
from jax.experimental.pallas import tpu_sc as plsc

N = 25000
FEAT = 128
H = 256
E = 400000
LBL = 100000

NC = 2
NS = 16
NW = NC * NS
ET = E // NS
ETP = 25600
NSB = 64

NSL = 2
SW = H // NSL
SPAD = 25600
NH = SPAD // 2
ACCR = NH + 128
CACC = 25728
TRASH = SPAD
CWID = 16

LTW = 3128
LPAD = LTW * NW
GBS = [112] * 27 + [104]

_mesh = lambda: plsc.VectorSubcoreMesh(
    core_axis_name="c", subcore_axis_name="s", num_cores=NC, num_subcores=NS)


def _enc_body(x_ref, w_ref, b_ref, emb_ref, *o_refs):
    o = (lax.dot_general(x_ref[...], w_ref[...], (((1,), (1,)), ((), ())),
                         preferred_element_type=jnp.float32)
         + b_ref[...] + emb_ref[...])
    for k in range(NSL):
        o_refs[k][...] = o[:, k * SW:(k + 1) * SW]


def _encode(x, w, b, emb):
    rb = 1000
    return pl.pallas_call(
        _enc_body,
        grid=(N // rb,),
        in_specs=[
            pl.BlockSpec((rb, FEAT), lambda i: (i, 0)),
            pl.BlockSpec((H, FEAT), lambda i: (0, 0)),
            pl.BlockSpec((1, H), lambda i: (0, 0)),
            pl.BlockSpec((rb, H), lambda i: (i, 0)),
        ],
        out_specs=[pl.BlockSpec((rb, SW), lambda i: (i, 0))] * NSL,
        out_shape=[jax.ShapeDtypeStruct((N, SW), jnp.float32)] * NSL,
    )(x, w, b.reshape(1, H), emb)


def _combine_body(*refs, relu, split):
    s_refs = refs[:NSL]
    c_ref = refs[NSL]
    x_refs = refs[NSL + 1:2 * NSL + 1]
    wl_ref, bl_ref, wr_ref = refs[2 * NSL + 1:2 * NSL + 4]
    o_refs = refs[2 * NSL + 4:]
    s = jnp.concatenate([r[...] for r in s_refs], axis=1)
    x = jnp.concatenate([r[...] for r in x_refs], axis=1)
    r = 1.0 / jnp.maximum(c_ref[..., 0:1], 1.0)
    agg = s * r
    o = (lax.dot_general(agg, wl_ref[...], (((1,), (1,)), ((), ())),
                         preferred_element_type=jnp.float32)
         + bl_ref[...]
         + lax.dot_general(x, wr_ref[...], (((1,), (1,)), ((), ())),
                           preferred_element_type=jnp.float32))
    if relu:
        o = jnp.maximum(o, 0.0)
    if split:
        for k in range(NSL):
            o_refs[k][...] = o[:, k * SW:(k + 1) * SW]
    else:
        o_refs[0][...] = o


def _combine(s_slices, c, x_slices, wl, bl, wr, relu, split):
    rb = 1000
    if split:
        out_specs = [pl.BlockSpec((rb, SW), lambda i: (i, 0))] * NSL
        out_shape = [jax.ShapeDtypeStruct((N, SW), jnp.float32)] * NSL
    else:
        out_specs = [pl.BlockSpec((rb, H), lambda i: (i, 0))]
        out_shape = [jax.ShapeDtypeStruct((N, H), jnp.float32)]
    return pl.pallas_call(
        functools.partial(_combine_body, relu=relu, split=split),
        grid=(N // rb,),
        in_specs=(
            [pl.BlockSpec((rb, SW), lambda i: (i, 0))] * NSL
            + [pl.BlockSpec((rb, SW), lambda i: (i, 0))]
            + [pl.BlockSpec((rb, SW), lambda i: (i, 0))] * NSL
            + [pl.BlockSpec((H, H), lambda i: (0, 0)),
               pl.BlockSpec((1, H), lambda i: (0, 0)),
               pl.BlockSpec((H, H), lambda i: (0, 0))]
        ),
        out_specs=out_specs,
        out_shape=out_shape,
    )(*s_slices, c, *x_slices, wl, bl.reshape(1, H), wr)


def _dot_body(a_ref, b_ref, o_ref):
    o_ref[...] = jnp.sum(a_ref[...] * b_ref[...], axis=1, keepdims=True)


def _edge_dot(a, b):
    rb = LTW
    return pl.pallas_call(
        _dot_body,
        grid=(LPAD // rb,),
        in_specs=[
            pl.BlockSpec((rb, H), lambda i: (i, 0)),
            pl.BlockSpec((rb, H), lambda i: (i, 0)),
        ],
        out_specs=pl.BlockSpec((rb, 1), lambda i: (i, 0)),
        out_shape=jax.ShapeDtypeStruct((LPAD, 1), jnp.float32),
    )(a, b)


def _zero2d(ref, nrows, ncols):
    zeros16 = jnp.zeros((16,), jnp.float32)
    for colblk in range(ncols // 16):
        def body(i, carry, _c=colblk):
            ref[i, pl.ds(_c * 16, 16)] = zeros16
            return carry
        lax.fori_loop(0, nrows, body, 0)


NJB = ETP // NSB


def _segsum_body(*refs):
    xs = refs[:NSL]
    g_hbm, s_hbm = refs[NSL], refs[NSL + 1]
    outs = refs[NSL + 2:2 * NSL + 2]
    (gstbig, sstbig, ssm0, ssm1, rows0, rows1, zb, acc,
     sem0, sem1) = refs[2 * NSL + 2:]
    cid = lax.axis_index("c")
    sid = lax.axis_index("s")
    ssm = (ssm0, ssm1)
    rows = (rows0, rows1)
    sems = (sem0, sem1)

    _zero2d(zb, 40, SW)
    eb = sid * ETP

    for f in range(NSL):
        for nh in range(2):
            def run_pass(_f=f, _nh=nh):
                xk = xs[_f]
                ok = outs[_f]
                base = _nh * NH
                r0 = sid * (ACCR // NS)
                off = 0
                for n in [40] * 20 + [8]:
                    pltpu.sync_copy(zb.at[pl.ds(0, n)],
                                    acc.at[pl.ds(r0 + off, n)])
                    off += n
                plsc.subcore_barrier()

                def localize(j):
                    for q in range(NSB // 16):
                        s = sstbig[pl.ds(j * NSB + q * 16, 16)]
                        loc = s - base
                        m = (loc >= 0) & (loc < NH)
                        ssm[j % 2][pl.ds(q * 16, 16)] = jnp.where(m, loc, NH)

                def blk(b2, carry):
                    b0 = eb + b2 * 16 * NSB
                    pltpu.sync_copy(g_hbm.at[pl.ds(b0, 16 * NSB)], gstbig)
                    pltpu.sync_copy(s_hbm.at[pl.ds(b0, 16 * NSB)], sstbig)
                    cp = pltpu.async_copy(
                        xk.at[gstbig.at[pl.ds(0, NSB)]], rows[0], sems[0])
                    for j in range(16):
                        localize(j)
                        cpn = None
                        if j + 1 < 16:
                            cpn = pltpu.async_copy(
                                xk.at[gstbig.at[pl.ds((j + 1) * NSB, NSB)]],
                                rows[(j + 1) % 2], sems[(j + 1) % 2])
                        cp.wait()
                        pltpu.sync_copy(rows[j % 2], acc.at[ssm[j % 2]],
                                        add=True)
                        cp = cpn
                    return carry
                lax.fori_loop(0, NJB // 16, blk, 0)
                plsc.subcore_barrier()
                w0 = sid * (NH // NS)
                pltpu.sync_copy(acc.at[pl.ds(w0, NH // NS)],
                                ok.at[pl.ds(base + w0, NH // NS)])
                plsc.subcore_barrier()
            pl.when(cid == f)(run_pass)


def _sc_segsum(x_slices, g1, s1):
    k = pl.kernel(
        _segsum_body,
        out_type=tuple(jax.ShapeDtypeStruct((SPAD, SW), jnp.float32)
                       for _ in range(NSL)),
        mesh=_mesh(),
        scratch_types=[
            pltpu.VMEM((16 * NSB,), jnp.int32),
            pltpu.VMEM((16 * NSB,), jnp.int32),
            pltpu.VMEM((NSB,), jnp.int32),
            pltpu.VMEM((NSB,), jnp.int32),
            pltpu.VMEM((NSB, SW), jnp.float32),
            pltpu.VMEM((NSB, SW), jnp.float32),
            pltpu.VMEM((40, SW), jnp.float32),
            pltpu.VMEM_SHARED((ACCR, SW), jnp.float32),
            pltpu.SemaphoreType.DMA,
            pltpu.SemaphoreType.DMA,
        ],
    )
    return k(*x_slices, g1, s1)


def _counts_body(su_hbm, sd_hbm, cu_hbm, cp_hbm,
                 sstbig, ssm0, ssm1, ones, zb, cnt):
    cid = lax.axis_index("c")
    sid = lax.axis_index("s")
    lanes = lax.iota(jnp.int32, 16)
    e0 = jnp.where(lanes == 0, 1.0, 0.0)
    zeros16 = jnp.zeros((16,), jnp.float32)
    ssm = (ssm0, ssm1)

    _zero2d(zb, 40, SW)

    def fill_ones(i, carry):
        ones[i, pl.ds(0, 16)] = e0
        for cb in range(1, SW // 16):
            ones[i, pl.ds(cb * 16, 16)] = zeros16
        return carry
    lax.fori_loop(0, NSB, fill_ones, 0)

    eb = sid * ETP

    for d in range(2):
        for nh in range(2):
            def run_pass(_d=d, _nh=nh):
                s_hbm = (su_hbm, sd_hbm)[_d]
                out_hbm = (cu_hbm, cp_hbm)[_d]
                base = _nh * NH
                r0 = sid * (ACCR // NS)
                off = 0
                for n in [40] * 20 + [8]:
                    pltpu.sync_copy(zb.at[pl.ds(0, n)],
                                    cnt.at[pl.ds(r0 + off, n)])
                    off += n
                plsc.subcore_barrier()

                def localize(j):
                    for q in range(NSB // 16):
                        s = sstbig[pl.ds(j * NSB + q * 16, 16)]
                        loc = s - base
                        m = (loc >= 0) & (loc < NH)
                        ssm[j % 2][pl.ds(q * 16, 16)] = jnp.where(m, loc, NH)

                def blk(b2, carry):
                    b0 = eb + b2 * 16 * NSB
                    pltpu.sync_copy(s_hbm.at[pl.ds(b0, 16 * NSB)], sstbig)
                    for j in range(16):
                        localize(j)
                        pltpu.sync_copy(ones, cnt.at[ssm[j % 2]], add=True)
                    return carry
                lax.fori_loop(0, NJB // 16, blk, 0)
                plsc.subcore_barrier()
                w0 = sid * (NH // NS)
                pltpu.sync_copy(cnt.at[pl.ds(w0, NH // NS)],
                                out_hbm.at[pl.ds(base + w0, NH // NS)])
                plsc.subcore_barrier()
            pl.when(cid == d)(run_pass)


def _sc_counts(s4u, s4p):
    k = pl.kernel(
        _counts_body,
        out_type=(jax.ShapeDtypeStruct((SPAD, SW), jnp.float32),
                  jax.ShapeDtypeStruct((SPAD, SW), jnp.float32)),
        mesh=_mesh(),
        scratch_types=[
            pltpu.VMEM((16 * NSB,), jnp.int32),
            pltpu.VMEM((NSB,), jnp.int32),
            pltpu.VMEM((NSB,), jnp.int32),
            pltpu.VMEM((NSB, SW), jnp.float32),
            pltpu.VMEM((40, SW), jnp.float32),
            pltpu.VMEM_SHARED((ACCR, SW), jnp.float32),
        ],
    )
    return k(s4u, s4p)


def _gather_body(hu_hbm, hp_hbm, i0_hbm, i1_hbm, eu_hbm, ep_hbm,
                 iv0, iv1, rows0, rows1, sem0, sem1):
    cid = lax.axis_index("c")
    sid = lax.axis_index("s")
    wid = cid * NS + sid
    base = wid * LTW
    pltpu.sync_copy(i0_hbm.at[pl.ds(base, LTW)], iv0)
    pltpu.sync_copy(i1_hbm.at[pl.ds(base, LTW)], iv1)
    boff = 0
    for gb in GBS:
        cp0 = pltpu.async_copy(hu_hbm.at[iv0.at[pl.ds(boff, gb)]],
                               rows0.at[pl.ds(0, gb)], sem0)
        cp1 = pltpu.async_copy(hp_hbm.at[iv1.at[pl.ds(boff, gb)]],
                               rows1.at[pl.ds(0, gb)], sem1)
        cp0.wait()
        cp1.wait()
        pltpu.sync_copy(rows0.at[pl.ds(0, gb)],
                        eu_hbm.at[pl.ds(base + boff, gb)])
        pltpu.sync_copy(rows1.at[pl.ds(0, gb)],
                        ep_hbm.at[pl.ds(base + boff, gb)])
        boff += gb


def _sc_gather2(hu, hp, i0, i1):
    k = pl.kernel(
        _gather_body,
        out_type=(jax.ShapeDtypeStruct((LPAD, H), jnp.float32),
                  jax.ShapeDtypeStruct((LPAD, H), jnp.float32)),
        mesh=_mesh(),
        scratch_types=[
            pltpu.VMEM((LTW,), jnp.int32),
            pltpu.VMEM((LTW,), jnp.int32),
            pltpu.VMEM((GBS[0], H), jnp.float32),
            pltpu.VMEM((GBS[0], H), jnp.float32),
            pltpu.SemaphoreType.DMA,
            pltpu.SemaphoreType.DMA,
        ],
    )
    return k(hu, hp, i0, i1)


def _edges4(v, pad):
    v = v.astype(jnp.int32).reshape(NS, ET)
    v = jnp.concatenate(
        [v, jnp.full((NS, ETP - ET), pad, jnp.int32)], axis=1)
    return v.reshape(-1)


def kernel(x_user, x_product, node_id_user, node_id_product, edge_index_up,
           edge_label_index, W_user_lin, b_user_lin, emb_user, W_prod_lin,
           b_prod_lin, emb_prod, Wl_up, bl_up, Wr_up, Wl_pu, bl_pu, Wr_pu):
    hu_s = _encode(x_user, W_user_lin, b_user_lin, emb_user)
    hp_s = _encode(x_product, W_prod_lin, b_prod_lin, emb_prod)

    src4 = _edges4(edge_index_up[0], TRASH)
    dst4 = _edges4(edge_index_up[1], TRASH)
    srcg = _edges4(edge_index_up[0], 0)
    dstg = _edges4(edge_index_up[1], 0)

    cu, cp = _sc_counts(src4, dst4)

    su_s = _sc_segsum(hp_s, dstg, src4)
    sp_s = _sc_segsum(hu_s, srcg, dst4)
    h1u_s = _combine(su_s, cu, hu_s, Wl_pu, bl_pu, Wr_pu, relu=True, split=True)
    h1p_s = _combine(sp_s, cp, hp_s, Wl_up, bl_up, Wr_up, relu=True, split=True)

    su2_s = _sc_segsum(h1p_s, dstg, src4)
    sp2_s = _sc_segsum(h1u_s, srcg, dst4)
    (h2u,) = _combine(su2_s, cu, h1u_s, Wl_pu, bl_pu, Wr_pu,
                      relu=False, split=False)
    (h2p,) = _combine(sp2_s, cp, h1p_s, Wl_up, bl_up, Wr_up,
                      relu=False, split=False)

    i0 = jnp.pad(edge_label_index[0].astype(jnp.int32), (0, LPAD - LBL))
    i1 = jnp.pad(edge_label_index[1].astype(jnp.int32), (0, LPAD - LBL))
    eu, ep = _sc_gather2(h2u, h2p, i0, i1)
    return _edge_dot(eu, ep).reshape(LPAD)[:LBL]
